# Initial kernel scaffold; baseline (speedup 1.0000x reference)
import jax
import jax.numpy as jnp
from jax.experimental import pallas as pl


def kernel(z, batch, W1, b1, W2, b2):
    raise NotImplementedError("write your pallas kernel here")


# speedup vs baseline: 1.2506x; 1.2506x over previous
import functools

import jax
import jax.numpy as jnp
from jax import lax
from jax.experimental import pallas as pl
from jax.experimental.pallas import tpu as pltpu
from jax.experimental.pallas import tpu_sc as plsc

_N = 160000
_D = 256
_NQ = 4
_QD = _D // _NQ
_S = 512
_CLS = 21
_C = 320
_NCHUNK = _N // _C
_NC, _NS = 2, 16
_NG = _NC * _NS // _NQ
_KMAX = -(-_NCHUNK // _NG)


def _sc_segment_sums(z, batch):
  mesh = plsc.VectorSubcoreMesh(core_axis_name="c", subcore_axis_name="s")

  @functools.partial(
      pl.kernel,
      out_type=(
          jax.ShapeDtypeStruct((_NG, _S, _D), jnp.float32),
          jax.ShapeDtypeStruct((_NG, _S, 16), jnp.float32),
      ),
      mesh=mesh,
      compiler_params=pltpu.CompilerParams(use_tc_tiling_on_sc=False),
      scratch_types=dict(
          idx_v=pltpu.VMEM((_C,), jnp.int32),
          rows_v=pltpu.VMEM((_C, _QD), jnp.float32),
          acc_v=pltpu.VMEM((_S, _QD), jnp.float32),
          cnt_v=pltpu.VMEM((_S, 16), jnp.float32),
      ),
  )
  def body(z_hbm, b_hbm, sums_hbm, cnts_hbm, idx_v, rows_v, acc_v, cnt_v):
    c = lax.axis_index("c")
    s = lax.axis_index("s")
    w = c * _NS + s
    grp = w // _NQ
    quarter = w % _NQ
    zeros16 = jnp.zeros((16,), jnp.float32)
    ones16 = jnp.full((16,), 1.0, jnp.float32)

    def zacc(i, _):
      for j in range(_QD // 16):
        acc_v[i, pl.ds(j * 16, 16)] = zeros16
      cnt_v[i, :] = zeros16
      return 0
    lax.fori_loop(0, _S, zacc, 0)

    def chunk(k, _):
      g = grp + _NG * k

      @pl.when(g < _NCHUNK)
      def _():
        pltpu.sync_copy(b_hbm.at[pl.ds(g * _C, _C)], idx_v)
        pltpu.sync_copy(
            z_hbm.at[pl.ds(g * _C, _C), pl.ds(quarter * _QD, _QD)], rows_v)

        def vec16(q, _):
          ids16 = idx_v[pl.ds(q * 16, 16)]
          for l in range(16):
            rid = ids16[l]
            r = q * 16 + l
            for j in range(_QD // 16):
              sl = pl.ds(j * 16, 16)
              acc_v[rid, sl] = acc_v[rid, sl] + rows_v[r, sl]
            cnt_v[rid, :] = cnt_v[rid, :] + ones16
          return 0
        lax.fori_loop(0, _C // 16, vec16, 0)
      return 0
    lax.fori_loop(0, _KMAX, chunk, 0)

    pltpu.sync_copy(acc_v, sums_hbm.at[grp, :, pl.ds(quarter * _QD, _QD)])

    @pl.when(quarter == 0)
    def _():
      pltpu.sync_copy(cnt_v, cnts_hbm.at[grp])

  return body(z, batch)


def _tc_head(sums, cnts, W1, b1, W2, b2):
  def body(s_ref, c_ref, w1_ref, b1_ref, w2_ref, b2_ref, o_ref):
    total = jnp.sum(s_ref[...], axis=0)
    counts = jnp.sum(c_ref[...], axis=0)[:, 0]
    mean = total / jnp.maximum(counts, 1.0)[:, None]
    h = lax.dot_general(mean, w1_ref[...], (((1,), (1,)), ((), ())),
                        preferred_element_type=jnp.float32) + b1_ref[...]
    h = jnp.maximum(h, 0.0)
    out = lax.dot_general(h, w2_ref[...], (((1,), (1,)), ((), ())),
                          preferred_element_type=jnp.float32) + b2_ref[...]
    o_ref[...] = out

  return pl.pallas_call(
      body,
      out_shape=jax.ShapeDtypeStruct((_S, _CLS), jnp.float32),
  )(sums, cnts, W1, b1.reshape(1, -1), W2, b2.reshape(1, -1))


def kernel(z, batch, W1, b1, W2, b2):
  batch = batch.astype(jnp.int32)
  sums, cnts = _sc_segment_sums(z, batch)
  return _tc_head(sums, cnts, W1, b1, W2, b2)


# speedup vs baseline: 2.0253x; 1.6194x over previous
import functools

import jax
import jax.numpy as jnp
from jax import lax
from jax.experimental import pallas as pl
from jax.experimental.pallas import tpu as pltpu
from jax.experimental.pallas import tpu_sc as plsc

_N = 160000
_D = 256
_NQ = 4
_QD = _D // _NQ
_S = 512
_CLS = 21
_C = 320
_NCHUNK = _N // _C
_NC, _NS = 2, 16
_NG = _NC * _NS // _NQ
_KMAX = -(-_NCHUNK // _NG)


def _sc_segment_sums(z, batch):
  mesh = plsc.VectorSubcoreMesh(core_axis_name="c", subcore_axis_name="s")

  @functools.partial(
      pl.kernel,
      out_type=(
          jax.ShapeDtypeStruct((_NG, _S, _D), jnp.float32),
          jax.ShapeDtypeStruct((_NG, _S, 16), jnp.float32),
      ),
      mesh=mesh,
      compiler_params=pltpu.CompilerParams(use_tc_tiling_on_sc=False),
      scratch_types=dict(
          idx_v=pltpu.VMEM((_C,), jnp.int32),
          rows_v=pltpu.VMEM((_C, _QD), jnp.float32),
          acc_v=pltpu.VMEM((_S, _QD), jnp.float32),
          cnt_v=pltpu.VMEM((_S, 16), jnp.float32),
          run_v=pltpu.VMEM((_QD // 16 + 1, 16), jnp.float32),
          cur_s=pltpu.SMEM((1,), jnp.int32),
      ),
  )
  def body(z_hbm, b_hbm, sums_hbm, cnts_hbm,
           idx_v, rows_v, acc_v, cnt_v, run_v, cur_s):
    c = lax.axis_index("c")
    s = lax.axis_index("s")
    w = c * _NS + s
    grp = w // _NQ
    quarter = w % _NQ
    zeros16 = jnp.zeros((16,), jnp.float32)
    ones16 = jnp.full((16,), 1.0, jnp.float32)

    def zacc(i, _):
      for j in range(_QD // 16):
        acc_v[i, pl.ds(j * 16, 16)] = zeros16
      cnt_v[i, :] = zeros16
      return 0
    lax.fori_loop(0, _S, zacc, 0)

    _NB = _QD // 16

    def flush(tgt):
      for j in range(_NB):
        acc_v[tgt, pl.ds(j * 16, 16)] = run_v[j, :]
      cnt_v[tgt, :] = run_v[_NB, :]

    cur_s[0] = jnp.int32(-1)
    for j in range(_NB + 1):
      run_v[j, :] = zeros16

    def chunk(k, _):
      g = grp + _NG * k
      pltpu.sync_copy(b_hbm.at[pl.ds(g * _C, _C)], idx_v)
      pltpu.sync_copy(
          z_hbm.at[pl.ds(g * _C, _C), pl.ds(quarter * _QD, _QD)], rows_v)

      def group(q, _):
        ids16 = idx_v[pl.ds(q * 16, 16)]
        first = ids16[0]
        last = ids16[15]
        r0 = q * 16
        cur = cur_s[0]
        fast = (first == cur) & (first == last)

        @pl.when(fast)
        def _():
          for j in range(_NB):
            sl = pl.ds(j * 16, 16)
            v = [rows_v[r0 + l, sl] for l in range(16)]
            while len(v) > 1:
              v = [v[i] + v[i + 1] for i in range(0, len(v) - 1, 2)] \
                  + ([v[-1]] if len(v) % 2 else [])
            run_v[j, :] = run_v[j, :] + v[0]
          run_v[_NB, :] = run_v[_NB, :] + jnp.full((16,), 16.0, jnp.float32)

        @pl.when(jnp.logical_not(fast))
        def _():
          for l in range(16):
            idl = ids16[l]
            cur_l = cur_s[0]

            @pl.when(idl != cur_l)
            def _():
              flush(jnp.maximum(cur_l, 0))
              for j in range(_NB):
                run_v[j, :] = rows_v[r0 + l, pl.ds(j * 16, 16)]
              run_v[_NB, :] = ones16
              cur_s[0] = idl

            @pl.when(idl == cur_l)
            def _():
              for j in range(_NB):
                run_v[j, :] = (run_v[j, :]
                               + rows_v[r0 + l, pl.ds(j * 16, 16)])
              run_v[_NB, :] = run_v[_NB, :] + ones16
        return 0

      lax.fori_loop(0, _C // 16, group, 0)
      return 0

    nk = (_NCHUNK - grp + _NG - 1) // _NG
    lax.fori_loop(0, nk, chunk, 0)
    flush(jnp.maximum(cur_s[0], 0))

    pltpu.sync_copy(acc_v, sums_hbm.at[grp, :, pl.ds(quarter * _QD, _QD)])

    @pl.when(quarter == 0)
    def _():
      pltpu.sync_copy(cnt_v, cnts_hbm.at[grp])

  return body(z, batch)


def _tc_head(sums, cnts, W1, b1, W2, b2):
  def body(s_ref, c_ref, w1_ref, b1_ref, w2_ref, b2_ref, o_ref):
    total = jnp.sum(s_ref[...], axis=0)
    counts = jnp.sum(c_ref[...], axis=0)[:, 0]
    mean = total / jnp.maximum(counts, 1.0)[:, None]
    h = lax.dot_general(mean, w1_ref[...], (((1,), (1,)), ((), ())),
                        preferred_element_type=jnp.float32) + b1_ref[...]
    h = jnp.maximum(h, 0.0)
    out = lax.dot_general(h, w2_ref[...], (((1,), (1,)), ((), ())),
                          preferred_element_type=jnp.float32) + b2_ref[...]
    o_ref[...] = out

  return pl.pallas_call(
      body,
      out_shape=jax.ShapeDtypeStruct((_S, _CLS), jnp.float32),
  )(sums, cnts, W1, b1.reshape(1, -1), W2, b2.reshape(1, -1))


def kernel(z, batch, W1, b1, W2, b2):
  batch = batch.astype(jnp.int32)
  sums, cnts = _sc_segment_sums(z, batch)
  return _tc_head(sums, cnts, W1, b1, W2, b2)


# speedup vs baseline: 2.7334x; 1.3497x over previous
import functools

import jax
import jax.numpy as jnp
from jax import lax
from jax.experimental import pallas as pl
from jax.experimental.pallas import tpu as pltpu
from jax.experimental.pallas import tpu_sc as plsc

_N = 160000
_D = 256
_NQ = 4
_QD = _D // _NQ
_S = 512
_CLS = 21
_C = 400
_NCHUNK = _N // _C
_NC, _NS = 2, 16
_NG = _NC * _NS // _NQ
_KPT = _NCHUNK // _NG


def _sc_segment_sums(z, batch):
  mesh = plsc.VectorSubcoreMesh(core_axis_name="c", subcore_axis_name="s")

  @functools.partial(
      pl.kernel,
      out_type=(
          jax.ShapeDtypeStruct((_NG, _S, _D), jnp.float32),
          jax.ShapeDtypeStruct((_NG, _S, 16), jnp.float32),
      ),
      mesh=mesh,
      compiler_params=pltpu.CompilerParams(use_tc_tiling_on_sc=False),
      scratch_types=dict(
          idx0=pltpu.VMEM((_C,), jnp.int32),
          idx1=pltpu.VMEM((_C,), jnp.int32),
          rows0=pltpu.VMEM((_C, _QD), jnp.float32),
          rows1=pltpu.VMEM((_C, _QD), jnp.float32),
          acc_v=pltpu.VMEM((_S, _QD), jnp.float32),
          cnt_v=pltpu.VMEM((_S, 16), jnp.float32),
          run_v=pltpu.VMEM((_QD // 16 + 1, 16), jnp.float32),
          cur_s=pltpu.SMEM((1,), jnp.int32),
          sem_r0=pltpu.SemaphoreType.DMA,
          sem_r1=pltpu.SemaphoreType.DMA,
          sem_i0=pltpu.SemaphoreType.DMA,
          sem_i1=pltpu.SemaphoreType.DMA,
      ),
  )
  def body(z_hbm, b_hbm, sums_hbm, cnts_hbm,
           idx0, idx1, rows0, rows1, acc_v, cnt_v, run_v, cur_s,
           sem_r0, sem_r1, sem_i0, sem_i1):
    c = lax.axis_index("c")
    s = lax.axis_index("s")
    w = c * _NS + s
    grp = w // _NQ
    quarter = w % _NQ
    zeros16 = jnp.zeros((16,), jnp.float32)
    ones16 = jnp.full((16,), 1.0, jnp.float32)

    def zacc(i, _):
      for j in range(_QD // 16):
        acc_v[i, pl.ds(j * 16, 16)] = zeros16
      cnt_v[i, :] = zeros16
      return 0
    lax.fori_loop(0, _S, zacc, 0)

    _NB = _QD // 16

    def flush(tgt):
      for j in range(_NB):
        acc_v[tgt, pl.ds(j * 16, 16)] = run_v[j, :]
      cnt_v[tgt, :] = run_v[_NB, :]

    cur_s[0] = jnp.int32(-1)
    for j in range(_NB + 1):
      run_v[j, :] = zeros16

    def zsrc(t):
      g = grp + _NG * t
      return z_hbm.at[pl.ds(g * _C, _C), pl.ds(quarter * _QD, _QD)]

    def bsrc(t):
      g = grp + _NG * t
      return b_hbm.at[pl.ds(g * _C, _C)]

    def start(t, rbuf, ibuf, rsem, isem):
      pltpu.async_copy(bsrc(t), ibuf, isem)
      pltpu.async_copy(zsrc(t), rbuf, rsem)

    def wait(t, rbuf, ibuf, rsem, isem):
      pltpu.make_async_copy(bsrc(t), ibuf, isem).wait()
      pltpu.make_async_copy(zsrc(t), rbuf, rsem).wait()

    def process(rows_v, idx_v):
      def group(q, _):
        ids16 = idx_v[pl.ds(q * 16, 16)]
        first = ids16[0]
        last = ids16[15]
        r0 = q * 16
        cur = cur_s[0]
        fast = (first == cur) & (first == last)

        @pl.when(fast)
        def _():
          for j in range(_NB):
            sl = pl.ds(j * 16, 16)
            v = [rows_v[r0 + l, sl] for l in range(16)]
            while len(v) > 1:
              v = [v[i] + v[i + 1] for i in range(0, len(v) - 1, 2)] \
                  + ([v[-1]] if len(v) % 2 else [])
            run_v[j, :] = run_v[j, :] + v[0]
          run_v[_NB, :] = run_v[_NB, :] + jnp.full((16,), 16.0, jnp.float32)

        @pl.when(jnp.logical_not(fast))
        def _():
          for l in range(16):
            idl = ids16[l]
            cur_l = cur_s[0]

            @pl.when(idl != cur_l)
            def _():
              flush(jnp.maximum(cur_l, 0))
              for j in range(_NB):
                run_v[j, :] = rows_v[r0 + l, pl.ds(j * 16, 16)]
              run_v[_NB, :] = ones16
              cur_s[0] = idl

            @pl.when(idl == cur_l)
            def _():
              for j in range(_NB):
                run_v[j, :] = (run_v[j, :]
                               + rows_v[r0 + l, pl.ds(j * 16, 16)])
              run_v[_NB, :] = run_v[_NB, :] + ones16
        return 0

      lax.fori_loop(0, _C // 16, group, 0)

    start(0, rows0, idx0, sem_r0, sem_i0)

    def pair(p, _):
      t0 = 2 * p
      start(t0 + 1, rows1, idx1, sem_r1, sem_i1)
      wait(t0, rows0, idx0, sem_r0, sem_i0)
      process(rows0, idx0)
      start(t0 + 2, rows0, idx0, sem_r0, sem_i0)
      wait(t0 + 1, rows1, idx1, sem_r1, sem_i1)
      process(rows1, idx1)
      return 0
    lax.fori_loop(0, _KPT // 2 - 1, pair, 0)

    start(_KPT - 1, rows1, idx1, sem_r1, sem_i1)
    wait(_KPT - 2, rows0, idx0, sem_r0, sem_i0)
    process(rows0, idx0)
    wait(_KPT - 1, rows1, idx1, sem_r1, sem_i1)
    process(rows1, idx1)
    flush(jnp.maximum(cur_s[0], 0))

    pltpu.sync_copy(acc_v, sums_hbm.at[grp, :, pl.ds(quarter * _QD, _QD)])

    @pl.when(quarter == 0)
    def _():
      pltpu.sync_copy(cnt_v, cnts_hbm.at[grp])

  return body(z, batch)


def _tc_head(sums, cnts, W1, b1, W2, b2):
  def body(s_ref, c_ref, w1_ref, b1_ref, w2_ref, b2_ref, o_ref):
    total = jnp.sum(s_ref[...], axis=0)
    counts = jnp.sum(c_ref[...], axis=0)[:, 0]
    mean = total / jnp.maximum(counts, 1.0)[:, None]
    h = lax.dot_general(mean, w1_ref[...], (((1,), (1,)), ((), ())),
                        preferred_element_type=jnp.float32) + b1_ref[...]
    h = jnp.maximum(h, 0.0)
    out = lax.dot_general(h, w2_ref[...], (((1,), (1,)), ((), ())),
                          preferred_element_type=jnp.float32) + b2_ref[...]
    o_ref[...] = out

  return pl.pallas_call(
      body,
      out_shape=jax.ShapeDtypeStruct((_S, _CLS), jnp.float32),
  )(sums, cnts, W1, b1.reshape(1, -1), W2, b2.reshape(1, -1))


def kernel(z, batch, W1, b1, W2, b2):
  batch = batch.astype(jnp.int32)
  sums, cnts = _sc_segment_sums(z, batch)
  return _tc_head(sums, cnts, W1, b1, W2, b2)
